# baseline (speedup 1.0000x reference)
import jax
import jax.numpy as jnp
from jax import lax
from jax.experimental import pallas as pl
from jax.experimental.pallas import tpu as pltpu
from jax.experimental.pallas import tpu_sc as plsc

B, C, H, W = 8, 384, 56, 56
P1 = C // 2
N = B * H * W
NW = 32
RPW = N // NW
R = 16
NCH = RPW // R
NK = P1 // 16


def _sc_body(x0, x1, bn1, bn2, thrh, o0, o1,
             bn1_v, bn2_v, thr_v,
             a0, a1, b0, b1, c0, c1, sem_in, sem_out):
    wid = lax.axis_index("s") * 2 + lax.axis_index("c")
    base = wid * RPW

    pltpu.sync_copy(bn1, bn1_v)
    pltpu.sync_copy(bn2, bn2_v)
    pltpu.sync_copy(thrh, thr_v)
    thr = thr_v[...]

    def fire_in(i, bx0, bx1):
        r0 = base + i * R
        pltpu.async_copy(x0.at[pl.ds(r0, R), :], bx0, sem_in)
        pltpu.async_copy(x1.at[pl.ds(r0, R), :], bx1, sem_in)

    def wait_in():
        pltpu.make_async_copy(x0.at[pl.ds(0, R), :], a0, sem_in).wait()
        pltpu.make_async_copy(x1.at[pl.ds(0, R), :], a1, sem_in).wait()

    def fire_out(i, bx0, bx1):
        r0 = base + i * R
        pltpu.async_copy(bx0, o0.at[pl.ds(r0, R), :], sem_out)
        pltpu.async_copy(bx1, o1.at[pl.ds(r0, R), :], sem_out)

    def wait_out():
        pltpu.make_async_copy(a0, o0.at[pl.ds(0, R), :], sem_out).wait()
        pltpu.make_async_copy(a1, o1.at[pl.ds(0, R), :], sem_out).wait()

    def compute(bx0, bx1):
        zero = jnp.zeros((16,), jnp.float32)
        for k in range(NK):
            cs = P1 + k * 16
            q1 = jnp.abs(bn1_v[pl.ds(cs, 16)])
            q2 = jnp.abs(bn2_v[pl.ds(cs, 16)])
            m0a = q1 > thr
            m0b = q1 < thr
            m1a = q2 > thr
            m1b = q2 < thr
            for r in range(R):
                x0c = bx0[r, pl.ds(cs, 16)]
                x1c = bx1[r, pl.ds(cs, 16)]
                bx0[r, pl.ds(cs, 16)] = jnp.where(
                    m0a, x0c, jnp.where(m0b, x1c, zero))
                bx1[r, pl.ds(cs, 16)] = jnp.where(
                    m1a, x1c, jnp.where(m1b, x0c, zero))

    def half(i, bx0, bx1, nx0, nx1):
        wait_in()
        compute(bx0, bx1)
        fire_out(i, bx0, bx1)

        @pl.when(i + 2 < NCH)
        def _():
            @pl.when(i >= 1)
            def _():
                wait_out()
            fire_in(i + 2, nx0, nx1)

    fire_in(0, a0, a1)
    fire_in(1, b0, b1)

    def body(p, carry):
        i0 = 3 * p
        half(i0, a0, a1, c0, c1)
        half(i0 + 1, b0, b1, a0, a1)
        half(i0 + 2, c0, c1, b0, b1)
        return carry

    lax.fori_loop(0, NCH // 3, body, 0)
    last = NCH - 1
    wait_in()
    compute(a0, a1)
    fire_out(last, a0, a1)
    wait_out()
    wait_out()
    wait_out()


@jax.jit
def _run(x0, x1, bn1, bn2, thr):
    x0r = x0.transpose(0, 2, 3, 1).reshape(N, C)
    x1r = x1.transpose(0, 2, 3, 1).reshape(N, C)
    thrh = jnp.full((16,), thr, jnp.float32)
    mesh = plsc.VectorSubcoreMesh(core_axis_name="c", subcore_axis_name="s")
    f = pl.kernel(
        _sc_body,
        out_type=[
            jax.ShapeDtypeStruct((N, C), jnp.float32),
            jax.ShapeDtypeStruct((N, C), jnp.float32),
        ],
        mesh=mesh,
        scratch_types=[
            pltpu.VMEM((C,), jnp.float32),
            pltpu.VMEM((C,), jnp.float32),
            pltpu.VMEM((16,), jnp.float32),
            pltpu.VMEM((R, C), jnp.float32),
            pltpu.VMEM((R, C), jnp.float32),
            pltpu.VMEM((R, C), jnp.float32),
            pltpu.VMEM((R, C), jnp.float32),
            pltpu.VMEM((R, C), jnp.float32),
            pltpu.VMEM((R, C), jnp.float32),
            pltpu.SemaphoreType.DMA,
            pltpu.SemaphoreType.DMA,
        ],
        compiler_params=pltpu.CompilerParams(use_tc_tiling_on_sc=True),
    )
    o0, o1 = f(x0r, x1r, bn1, bn2, thrh)
    o0 = o0.reshape(B, H, W, C).transpose(0, 3, 1, 2)
    o1 = o1.reshape(B, H, W, C).transpose(0, 3, 1, 2)
    return o0, o1


def kernel(x0, x1, bn1_weight, bn2_weight, bn_threshold):
    return _run(x0, x1, bn1_weight, bn2_weight, bn_threshold)
